# baseline (speedup 1.0000x reference)
import functools

import numpy as np
import jax
import jax.numpy as jnp
from jax import lax
from jax.experimental import pallas as pl
from jax.experimental.pallas import tpu as pltpu
from jax.experimental.pallas import tpu_sc as plsc

_RADIUS = 0.1
_NSAMPLE = 32
_L = 16
_NC, _NS = 2, 16
_NW = _NC * _NS


@functools.lru_cache(maxsize=None)
def _build(B, N, NP, C):
    CPW = (B * NP) // _NW
    NCH = N // _L
    OUTW = CPW * _NSAMPLE
    VPW = OUTW // _L
    WPB = NP // CPW
    OUTROW = NP * _NSAMPLE
    R2 = np.float32(_RADIUS * _RADIUS)

    mesh = plsc.VectorSubcoreMesh(core_axis_name="c", subcore_axis_name="s",
                                  num_cores=_NC, num_subcores=_NS)

    @functools.partial(
        pl.kernel,
        out_type=jax.ShapeDtypeStruct((B * (C + 3) * _NSAMPLE, NP),
                                      jnp.float32),
        mesh=mesh,
        compiler_params=pltpu.CompilerParams(needs_layout_passes=False),
        scratch_types=[
            pltpu.VMEM((3 * N,), jnp.float32),
            pltpu.VMEM((3 * CPW,), jnp.float32),
            pltpu.VMEM((CPW * _NSAMPLE + N + _L,), jnp.int32),
            pltpu.VMEM((2 * N,), jnp.float32),
            pltpu.VMEM((_NSAMPLE, CPW), jnp.float32),
            pltpu.VMEM((_NSAMPLE, CPW), jnp.float32),
            pltpu.SemaphoreType.DMA,
            pltpu.SemaphoreType.DMA,
            pltpu.SemaphoreType.DMA,
            pltpu.SemaphoreType.DMA,
        ],
    )
    def sc_kernel(xyzT, nxT, feat, out, pts, cents, stage, frow, ostage,
                  ostage1, semi0, semi1, semo0, semo1):
        wid = lax.axis_index("s") * _NC + lax.axis_index("c")
        b = wid // WPB
        p_base = (wid % WPB) * CPW
        xyz_base = b * 3 * N
        nx_base = b * 3 * NP + p_base
        feat_base = b * C * N
        out_row = b * (C + 3) * _NSAMPLE
        for d in range(3):
            pltpu.sync_copy(xyzT.at[pl.ds(xyz_base + d * N, N)],
                            pts.at[pl.ds(d * N, N)])
            pltpu.sync_copy(nxT.at[pl.ds(nx_base + d * NP, CPW)],
                            cents.at[pl.ds(d * CPW, CPW)])
        iot = lax.iota(jnp.int32, _L)

        def per_centroid(p, carry):
            pfull = jnp.full((_L,), p, jnp.int32)
            cx = plsc.load_gather(cents, [pfull])
            cy = plsc.load_gather(cents, [pfull + CPW])
            cz = plsc.load_gather(cents, [pfull + 2 * CPW])
            row = p * _NSAMPLE

            def chunk_body(i, found):
                base = i * _L
                xv = pts[pl.ds(base, _L)]
                yv = pts[pl.ds(N + base, _L)]
                zv = pts[pl.ds(2 * N + base, _L)]
                dx = cx - xv
                dy = cy - yv
                dz = cz - zv
                d2 = (dx * dx + dy * dy) + dz * dz
                hit = d2 < R2
                plsc.store_compressed(stage.at[pl.ds(row + found, _L)],
                                      iot + base, mask=hit)
                cntv = plsc.all_reduce_population_count(hit)
                return found + lax.squeeze(lax.slice(cntv, (0,), (1,)), (0,))

            found = plsc.parallel_loop(0, NCH, unroll=4,
                                       carry=jnp.int32(0))(chunk_body)

            firstv = plsc.load_gather(stage, [jnp.full((_L,), row, jnp.int32)])
            firstv = jnp.where(found > 0, firstv, 0)
            foundv = jnp.full((_L,), found)
            for half in range(_NSAMPLE // _L):
                pos = iot + half * _L
                cur = stage[pl.ds(row + half * _L, _L)]
                stage[pl.ds(row + half * _L, _L)] = jnp.where(pos < foundv,
                                                              cur, firstv)
            return carry

        lax.fori_loop(0, CPW, per_centroid, jnp.int32(0))

        for d in range(3):
            def per_cent_xyz(p, d=d):
                cb = plsc.load_gather(
                    cents, [jnp.full((_L,), d * CPW + p, jnp.int32)])
                pvec = jnp.full((_L,), p, jnp.int32)
                for half in range(_NSAMPLE // _L):
                    i = p * (_NSAMPLE // _L) + half
                    idxv = stage[pl.ds(i * _L, _L)]
                    g = plsc.load_gather(pts, [idxv + d * N])
                    plsc.store_scatter(ostage, [iot + half * _L, pvec],
                                       g - cb)

            plsc.parallel_loop(0, CPW, unroll=4)(per_cent_xyz)
            pltpu.sync_copy(
                ostage,
                out.at[pl.ds((out_row + d * _NSAMPLE), _NSAMPLE),
                       pl.ds(p_base, CPW)])

        iot16 = iot + _L

        def gather_into(par):
            ost = ostage if par == 0 else ostage1

            def per_p(p):
                pv = jnp.full((_L,), p, jnp.int32)
                for h in range(2):
                    idxv = stage[pl.ds((2 * p + h) * _L, _L)]
                    if par:
                        idxv = idxv + N
                    g = plsc.load_gather(frow, [idxv])
                    plsc.store_scatter(ost, [iot if h == 0 else iot16, pv], g)
            plsc.parallel_loop(0, CPW, unroll=8)(per_p)

        pltpu.async_copy(feat.at[pl.ds(feat_base, N)],
                         frow.at[pl.ds(0, N)], semi0)
        pltpu.async_copy(feat.at[pl.ds(feat_base + N, N)],
                         frow.at[pl.ds(N, N)], semi1)

        def per_pair(j, carry):
            for par in range(2):
                c = 2 * j + par
                semi = semi0 if par == 0 else semi1
                semo = semo0 if par == 0 else semo1
                pltpu.make_async_copy(feat.at[pl.ds(feat_base + c * N, N)],
                                      frow.at[pl.ds(par * N, N)], semi).wait()

                ost = ostage if par == 0 else ostage1

                @pl.when(j > 0)
                def _drain():
                    pltpu.make_async_copy(
                        ost,
                        out.at[pl.ds(out_row, _NSAMPLE), pl.ds(p_base, CPW)],
                        semo).wait()

                gather_into(par)
                pltpu.async_copy(
                    ost,
                    out.at[pl.ds(out_row + (3 + c) * _NSAMPLE, _NSAMPLE),
                           pl.ds(p_base, CPW)], semo)
                cc = jnp.minimum(c + 2, C - 1)
                pltpu.async_copy(feat.at[pl.ds(feat_base + cc * N, N)],
                                 frow.at[pl.ds(par * N, N)], semi)
            return carry

        lax.fori_loop(0, C // 2, per_pair, jnp.int32(0))

        pltpu.make_async_copy(feat.at[pl.ds(feat_base, N)],
                              frow.at[pl.ds(0, N)], semi0).wait()
        pltpu.make_async_copy(feat.at[pl.ds(feat_base, N)],
                              frow.at[pl.ds(N, N)], semi1).wait()
        pltpu.make_async_copy(ostage,
                              out.at[pl.ds(out_row, _NSAMPLE),
                                     pl.ds(p_base, CPW)], semo0).wait()
        pltpu.make_async_copy(ostage1,
                              out.at[pl.ds(out_row, _NSAMPLE),
                                     pl.ds(p_base, CPW)], semo1).wait()

    return sc_kernel


def kernel(xyz, new_xyz, features):
    B, N, _ = xyz.shape
    NP = new_xyz.shape[1]
    C = features.shape[1]
    xyzT = jnp.transpose(xyz, (0, 2, 1)).reshape(-1)
    nxT = jnp.transpose(new_xyz, (0, 2, 1)).reshape(-1)
    out = _build(B, N, NP, C)(xyzT, nxT, features.reshape(-1))
    out = out.reshape(B, C + 3, _NSAMPLE, NP)
    return jnp.transpose(out, (0, 1, 3, 2))


# speedup vs baseline: 1.0019x; 1.0019x over previous
import functools

import numpy as np
import jax
import jax.numpy as jnp
from jax import lax
from jax.experimental import pallas as pl
from jax.experimental.pallas import tpu as pltpu
from jax.experimental.pallas import tpu_sc as plsc

_RADIUS = 0.1
_NSAMPLE = 32
_L = 16
_NC, _NS = 2, 16
_NW = _NC * _NS


@functools.lru_cache(maxsize=None)
def _build(B, N, NP, C):
    CPW = (B * NP) // _NW
    NCH = N // _L
    OUTW = CPW * _NSAMPLE
    VPW = OUTW // _L
    WPB = NP // CPW
    OUTROW = NP * _NSAMPLE
    R2 = np.float32(_RADIUS * _RADIUS)

    mesh = plsc.VectorSubcoreMesh(core_axis_name="c", subcore_axis_name="s",
                                  num_cores=_NC, num_subcores=_NS)

    @functools.partial(
        pl.kernel,
        out_type=jax.ShapeDtypeStruct((B * (C + 3) * _NSAMPLE, NP),
                                      jnp.float32),
        mesh=mesh,
        compiler_params=pltpu.CompilerParams(needs_layout_passes=False),
        scratch_types=[
            pltpu.VMEM((3 * N,), jnp.float32),
            pltpu.VMEM((3 * CPW,), jnp.float32),
            pltpu.VMEM((CPW * _NSAMPLE + N + _L,), jnp.int32),
            pltpu.VMEM((2 * N,), jnp.float32),
            pltpu.VMEM((_NSAMPLE, CPW), jnp.float32),
            pltpu.VMEM((_NSAMPLE, CPW), jnp.float32),
            pltpu.SemaphoreType.DMA,
            pltpu.SemaphoreType.DMA,
            pltpu.SemaphoreType.DMA,
            pltpu.SemaphoreType.DMA,
        ],
    )
    def sc_kernel(xyzT, nxT, feat, out, pts, cents, stage, frow, ostage,
                  ostage1, semi0, semi1, semo0, semo1):
        wid = lax.axis_index("s") * _NC + lax.axis_index("c")
        b = wid // WPB
        p_base = (wid % WPB) * CPW
        xyz_base = b * 3 * N
        nx_base = b * 3 * NP + p_base
        feat_base = b * C * N
        out_row = b * (C + 3) * _NSAMPLE
        for d in range(3):
            pltpu.sync_copy(xyzT.at[pl.ds(xyz_base + d * N, N)],
                            pts.at[pl.ds(d * N, N)])
            pltpu.sync_copy(nxT.at[pl.ds(nx_base + d * NP, CPW)],
                            cents.at[pl.ds(d * CPW, CPW)])
        iot = lax.iota(jnp.int32, _L)

        def per_centroid(p, carry):
            pfull = jnp.full((_L,), p, jnp.int32)
            cx = plsc.load_gather(cents, [pfull])
            cy = plsc.load_gather(cents, [pfull + CPW])
            cz = plsc.load_gather(cents, [pfull + 2 * CPW])
            row = p * _NSAMPLE

            def chunk_body(i, found):
                base = i * _L
                xv = pts[pl.ds(base, _L)]
                yv = pts[pl.ds(N + base, _L)]
                zv = pts[pl.ds(2 * N + base, _L)]
                dx = cx - xv
                dy = cy - yv
                dz = cz - zv
                d2 = (dx * dx + dy * dy) + dz * dz
                hit = d2 < R2
                plsc.store_compressed(stage.at[pl.ds(row + found, _L)],
                                      iot + base, mask=hit)
                cntv = plsc.all_reduce_population_count(hit)
                return found + lax.squeeze(lax.slice(cntv, (0,), (1,)), (0,))

            found = plsc.parallel_loop(0, NCH, unroll=4,
                                       carry=jnp.int32(0))(chunk_body)

            firstv = plsc.load_gather(stage, [jnp.full((_L,), row, jnp.int32)])
            firstv = jnp.where(found > 0, firstv, 0)
            foundv = jnp.full((_L,), found)
            for half in range(_NSAMPLE // _L):
                pos = iot + half * _L
                cur = stage[pl.ds(row + half * _L, _L)]
                stage[pl.ds(row + half * _L, _L)] = jnp.where(pos < foundv,
                                                              cur, firstv)
            return carry

        lax.fori_loop(0, CPW, per_centroid, jnp.int32(0))

        for d in range(3):
            def per_cent_xyz(p, d=d):
                cb = plsc.load_gather(
                    cents, [jnp.full((_L,), d * CPW + p, jnp.int32)])
                pvec = jnp.full((_L,), p, jnp.int32)
                for half in range(_NSAMPLE // _L):
                    i = p * (_NSAMPLE // _L) + half
                    idxv = stage[pl.ds(i * _L, _L)]
                    g = plsc.load_gather(pts, [idxv + d * N])
                    plsc.store_scatter(ostage, [iot + half * _L, pvec],
                                       g - cb)

            plsc.parallel_loop(0, CPW, unroll=2)(per_cent_xyz)
            pltpu.sync_copy(
                ostage,
                out.at[pl.ds((out_row + d * _NSAMPLE), _NSAMPLE),
                       pl.ds(p_base, CPW)])

        iot16 = iot + _L

        def gather_into(par):
            ost = ostage if par == 0 else ostage1

            def per_p(p):
                pv = jnp.full((_L,), p, jnp.int32)
                for h in range(2):
                    idxv = stage[pl.ds((2 * p + h) * _L, _L)]
                    if par:
                        idxv = idxv + N
                    g = plsc.load_gather(frow, [idxv])
                    plsc.store_scatter(ost, [iot if h == 0 else iot16, pv], g)
            plsc.parallel_loop(0, CPW, unroll=4)(per_p)

        pltpu.async_copy(feat.at[pl.ds(feat_base, N)],
                         frow.at[pl.ds(0, N)], semi0)
        pltpu.async_copy(feat.at[pl.ds(feat_base + N, N)],
                         frow.at[pl.ds(N, N)], semi1)

        def per_pair(j, carry):
            for par in range(2):
                c = 2 * j + par
                semi = semi0 if par == 0 else semi1
                semo = semo0 if par == 0 else semo1
                pltpu.make_async_copy(feat.at[pl.ds(feat_base + c * N, N)],
                                      frow.at[pl.ds(par * N, N)], semi).wait()

                ost = ostage if par == 0 else ostage1

                @pl.when(j > 0)
                def _drain():
                    pltpu.make_async_copy(
                        ost,
                        out.at[pl.ds(out_row, _NSAMPLE), pl.ds(p_base, CPW)],
                        semo).wait()

                gather_into(par)
                pltpu.async_copy(
                    ost,
                    out.at[pl.ds(out_row + (3 + c) * _NSAMPLE, _NSAMPLE),
                           pl.ds(p_base, CPW)], semo)
                cc = jnp.minimum(c + 2, C - 1)
                pltpu.async_copy(feat.at[pl.ds(feat_base + cc * N, N)],
                                 frow.at[pl.ds(par * N, N)], semi)
            return carry

        lax.fori_loop(0, C // 2, per_pair, jnp.int32(0))

        pltpu.make_async_copy(feat.at[pl.ds(feat_base, N)],
                              frow.at[pl.ds(0, N)], semi0).wait()
        pltpu.make_async_copy(feat.at[pl.ds(feat_base, N)],
                              frow.at[pl.ds(N, N)], semi1).wait()
        pltpu.make_async_copy(ostage,
                              out.at[pl.ds(out_row, _NSAMPLE),
                                     pl.ds(p_base, CPW)], semo0).wait()
        pltpu.make_async_copy(ostage1,
                              out.at[pl.ds(out_row, _NSAMPLE),
                                     pl.ds(p_base, CPW)], semo1).wait()

    return sc_kernel


def kernel(xyz, new_xyz, features):
    B, N, _ = xyz.shape
    NP = new_xyz.shape[1]
    C = features.shape[1]
    xyzT = jnp.transpose(xyz, (0, 2, 1)).reshape(-1)
    nxT = jnp.transpose(new_xyz, (0, 2, 1)).reshape(-1)
    out = _build(B, N, NP, C)(xyzT, nxT, features.reshape(-1))
    out = out.reshape(B, C + 3, _NSAMPLE, NP)
    return jnp.transpose(out, (0, 1, 3, 2))
